# Initial kernel scaffold; baseline (speedup 1.0000x reference)
import jax
import jax.numpy as jnp
from jax.experimental import pallas as pl


def kernel(x, pe_w1, pe_b1, pe_w2, pe_b2, gate_w, gate_b, exp_w1, exp_b1, exp_w2, exp_b2, ln_g, ln_b, cls_w1, cls_b1, cls_w2, cls_b2):
    raise NotImplementedError("write your pallas kernel here")


# speedup vs baseline: 1.9474x; 1.9474x over previous
import functools

import jax
import jax.numpy as jnp
from jax.experimental import pallas as pl
from jax.experimental.pallas import tpu as pltpu

B, S, DIN, D, HID, E, NCLS = 64, 196, 768, 64, 256, 4, 10
T = B * S
BT = 784
NBLK = T // BT
BATCHES_PER_BLK = BT // S


def _gelu(v):
    return 0.5 * v * (1.0 + jax.lax.erf(v * 0.7071067811865476))


def _fused_body(x_ref, pe_w1_ref, pe_b1_ref, pe_w2_ref, pe_b2_ref,
                gate_w_ref, gate_b_ref, exp_w1_ref, exp_b1_ref,
                exp_w2_ref, exp_b2_ref, ln_g_ref, ln_b_ref,
                cls_w1_ref, cls_b1_ref, cls_w2_ref, cls_b2_ref,
                logits_ref, moe_ref, aux_ref,
                pooled_acc, imp_acc, cnt_acc):
    i = pl.program_id(0)

    @pl.when(i == 0)
    def _():
        imp_acc[...] = jnp.zeros_like(imp_acc)
        cnt_acc[...] = jnp.zeros_like(cnt_acc)

    xb = x_ref[...]
    h1 = _gelu(jnp.dot(xb, pe_w1_ref[...], preferred_element_type=jnp.float32)
               + pe_b1_ref[...])
    tok = (jnp.dot(h1, pe_w2_ref[...], preferred_element_type=jnp.float32)
           + pe_b2_ref[...])

    glog = (jnp.dot(tok, gate_w_ref[...], preferred_element_type=jnp.float32)
            + gate_b_ref[...])
    m = jnp.max(glog, axis=-1, keepdims=True)
    p = jnp.exp(glog - m)
    probs = p / jnp.sum(p, axis=-1, keepdims=True)
    gval = jnp.max(probs, axis=-1, keepdims=True)
    idx = jnp.argmax(probs, axis=-1).reshape(BT, 1)
    eids = jax.lax.broadcasted_iota(jnp.int32, (BT, E), 1)
    oh = (idx == eids).astype(jnp.float32)

    imp_acc[...] += jnp.sum(probs, axis=0, keepdims=True)
    cnt_acc[...] += jnp.sum(oh, axis=0, keepdims=True)

    moe = jnp.zeros((BT, D), dtype=jnp.float32)
    for e in range(E):
        he = _gelu(jnp.dot(tok, exp_w1_ref[e],
                           preferred_element_type=jnp.float32)
                   + exp_b1_ref[e][None, :])
        oe = (jnp.dot(he, exp_w2_ref[e], preferred_element_type=jnp.float32)
              + exp_b2_ref[e][None, :])
        moe += (oh[:, e:e + 1] * gval) * oe

    moe3 = moe.reshape(BATCHES_PER_BLK, S, D)
    moe_ref[...] = moe3
    pooled_acc[pl.ds(i * BATCHES_PER_BLK, BATCHES_PER_BLK), :] = (
        jnp.mean(moe3, axis=1))

    @pl.when(i == NBLK - 1)
    def _():
        pooled = pooled_acc[...]
        mu = jnp.mean(pooled, axis=-1, keepdims=True)
        var = jnp.mean((pooled - mu) ** 2, axis=-1, keepdims=True)
        ln = ((pooled - mu) / jnp.sqrt(var + 1e-5) * ln_g_ref[...]
              + ln_b_ref[...])
        c = _gelu(jnp.dot(ln, cls_w1_ref[...],
                          preferred_element_type=jnp.float32)
                  + cls_b1_ref[...])
        logits_ref[...] = (jnp.dot(c, cls_w2_ref[...],
                                   preferred_element_type=jnp.float32)
                           + cls_b2_ref[...])
        imp = imp_acc[...] / T
        load = cnt_acc[...] / T
        aux = E * jnp.sum(imp * load)
        aux_ref[...] = jnp.full((1, 128), aux, dtype=jnp.float32)


def kernel(x, pe_w1, pe_b1, pe_w2, pe_b2, gate_w, gate_b, exp_w1, exp_b1,
           exp_w2, exp_b2, ln_g, ln_b, cls_w1, cls_b1, cls_w2, cls_b2,
           interpret=False):
    xr = x.reshape(T, DIN)

    full = lambda shape: pl.BlockSpec(shape, lambda i: (0,) * len(shape))
    logits, moe_out, aux = pl.pallas_call(
        _fused_body,
        grid=(NBLK,),
        in_specs=[
            pl.BlockSpec((BT, DIN), lambda i: (i, 0)),
            full((DIN, D)),
            full((1, D)),
            full((D, D)),
            full((1, D)),
            full((D, E)),
            full((1, E)),
            full((E, D, HID)),
            full((E, HID)),
            full((E, HID, D)),
            full((E, D)),
            full((1, D)),
            full((1, D)),
            full((D, D)),
            full((1, D)),
            full((D, NCLS)),
            full((1, NCLS)),
        ],
        out_specs=[
            pl.BlockSpec((B, NCLS), lambda i: (0, 0)),
            pl.BlockSpec((BATCHES_PER_BLK, S, D), lambda i: (i, 0, 0)),
            pl.BlockSpec((1, 128), lambda i: (0, 0)),
        ],
        out_shape=[
            jax.ShapeDtypeStruct((B, NCLS), jnp.float32),
            jax.ShapeDtypeStruct((B, S, D), jnp.float32),
            jax.ShapeDtypeStruct((1, 128), jnp.float32),
        ],
        scratch_shapes=[
            pltpu.VMEM((B, D), jnp.float32),
            pltpu.VMEM((1, E), jnp.float32),
            pltpu.VMEM((1, E), jnp.float32),
        ],
        interpret=interpret,
    )(xr, pe_w1, pe_b1.reshape(1, D), pe_w2, pe_b2.reshape(1, D),
      gate_w, gate_b.reshape(1, E), exp_w1, exp_b1, exp_w2, exp_b2,
      ln_g.reshape(1, D), ln_b.reshape(1, D), cls_w1,
      cls_b1.reshape(1, D), cls_w2, cls_b2.reshape(1, NCLS))
    return logits, moe_out, aux[0, 0]
